# baseline (device time: 21449 ns/iter reference)
import jax
import jax.numpy as jnp
from jax import lax
from jax.experimental import pallas as pl
from jax.experimental.pallas import tpu as pltpu

N_DEV = 8
MASKS = (1, 3, 4)
EXPERTS_PER_DEV = 2
C = 4


def kernel(x, router_W, route_idx, expert_W):
    del router_W
    n, d = x.shape
    h = expert_W.shape[-1]
    cw = h // C

    def body(x_ref, idx_ref, ew_ref, out_ref, acc_ref, comm_ref,
             send_sems, recv_sems):
        my = lax.axis_index("i")

        barrier_sem = pltpu.get_barrier_semaphore()
        for m in MASKS:
            pl.semaphore_signal(
                barrier_sem, inc=1,
                device_id=(my ^ m,),
                device_id_type=pl.DeviceIdType.MESH,
            )

        def mk(k, c):
            return pltpu.make_async_remote_copy(
                src_ref=acc_ref.at[c],
                dst_ref=comm_ref.at[k, c],
                send_sem=send_sems.at[k, c],
                recv_sem=recv_sems.at[k, c],
                device_id=(my ^ MASKS[k],),
                device_id_type=pl.DeviceIdType.MESH,
            )

        idx = idx_ref[:, :]
        descs = {}

        for c in range(C):
            a = jnp.zeros((n, cw), jnp.float32)
            for e in range(EXPERTS_PER_DEV):
                ge = my * EXPERTS_PER_DEV + e
                y = jnp.dot(x_ref[:, :], ew_ref[e, :, c * cw:(c + 1) * cw],
                            preferred_element_type=jnp.float32)
                a = a + jnp.where(idx == ge, y, 0.0)
            acc_ref[c] = a
            if c == 0:
                pl.semaphore_wait(barrier_sem, len(MASKS))
            dsc = mk(0, c)
            dsc.start()
            descs[(0, c)] = dsc

        for k in range(len(MASKS)):
            for c in range(C):
                descs[(k, c)].wait()
                if k < len(MASKS) - 1:
                    acc_ref[c] = acc_ref[c] + comm_ref[k, c]
                    dsc = mk(k + 1, c)
                    dsc.start()
                    descs[(k + 1, c)] = dsc
                else:
                    out_ref[:, c * cw:(c + 1) * cw] = (
                        acc_ref[c] + comm_ref[k, c])

    return pl.pallas_call(
        body,
        out_shape=jax.ShapeDtypeStruct((n, h), jnp.float32),
        in_specs=[
            pl.BlockSpec(memory_space=pltpu.VMEM),
            pl.BlockSpec(memory_space=pltpu.VMEM),
            pl.BlockSpec(memory_space=pltpu.VMEM),
        ],
        out_specs=pl.BlockSpec(memory_space=pltpu.VMEM),
        scratch_shapes=[
            pltpu.VMEM((C, n, cw), jnp.float32),
            pltpu.VMEM((len(MASKS), C, n, cw), jnp.float32),
            pltpu.SemaphoreType.DMA((len(MASKS), C)),
            pltpu.SemaphoreType.DMA((len(MASKS), C)),
        ],
        compiler_params=pltpu.CompilerParams(collective_id=0),
    )(x, route_idx, expert_W)


# device time: 14988 ns/iter; 1.4311x vs baseline; 1.4311x over previous
import jax
import jax.numpy as jnp
from jax import lax
from jax.experimental import pallas as pl
from jax.experimental.pallas import tpu as pltpu

N_DEV = 8
EXPERTS_PER_DEV = 2


def kernel(x, router_W, route_idx, expert_W):
    del router_W
    n, d = x.shape
    h = expert_W.shape[-1]
    B = n // N_DEV

    def body(x_ref, idx_ref, ew_ref, out_ref, acc_ref, a2a_ref,
             b_ssems, b_rsems, c_ssems, c_rsems):
        my = lax.axis_index("i")

        barrier_sem = pltpu.get_barrier_semaphore()
        for j in range(1, N_DEV):
            pl.semaphore_signal(
                barrier_sem, inc=1,
                device_id=((my + j) % N_DEV,),
                device_id_type=pl.DeviceIdType.MESH,
            )

        idx = idx_ref[:, :]
        acc = jnp.zeros((n, h), jnp.float32)
        for e in range(EXPERTS_PER_DEV):
            ge = my * EXPERTS_PER_DEV + e
            y = jnp.dot(x_ref[:, :], ew_ref[e, :, :],
                        preferred_element_type=jnp.float32)
            acc = acc + jnp.where(idx == ge, y, 0.0)
        acc_ref[:, :] = acc

        pl.semaphore_wait(barrier_sem, N_DEV - 1)

        bdescs = []
        for j in range(1, N_DEV):
            r = (my + j) % N_DEV
            dsc = pltpu.make_async_remote_copy(
                src_ref=acc_ref.at[pl.ds(r * B, B), :],
                dst_ref=a2a_ref.at[j - 1],
                send_sem=b_ssems.at[j - 1],
                recv_sem=b_rsems.at[j - 1],
                device_id=(r,),
                device_id_type=pl.DeviceIdType.MESH,
            )
            dsc.start()
            bdescs.append(dsc)
        for dsc in bdescs:
            dsc.wait()

        blk = acc_ref[pl.ds(my * B, B), :]
        for j in range(N_DEV - 1):
            blk = blk + a2a_ref[j]
        out_ref[pl.ds(my * B, B), :] = blk

        cdescs = []
        for j in range(1, N_DEV):
            r = (my + j) % N_DEV
            dsc = pltpu.make_async_remote_copy(
                src_ref=out_ref.at[pl.ds(my * B, B), :],
                dst_ref=out_ref.at[pl.ds(my * B, B), :],
                send_sem=c_ssems.at[j - 1],
                recv_sem=c_rsems.at[j - 1],
                device_id=(r,),
                device_id_type=pl.DeviceIdType.MESH,
            )
            dsc.start()
            cdescs.append(dsc)
        for dsc in cdescs:
            dsc.wait()

    return pl.pallas_call(
        body,
        out_shape=jax.ShapeDtypeStruct((n, h), jnp.float32),
        in_specs=[
            pl.BlockSpec(memory_space=pltpu.VMEM),
            pl.BlockSpec(memory_space=pltpu.VMEM),
            pl.BlockSpec(memory_space=pltpu.VMEM),
        ],
        out_specs=pl.BlockSpec(memory_space=pltpu.VMEM),
        scratch_shapes=[
            pltpu.VMEM((n, h), jnp.float32),
            pltpu.VMEM((N_DEV - 1, B, h), jnp.float32),
            pltpu.SemaphoreType.DMA((N_DEV - 1,)),
            pltpu.SemaphoreType.DMA((N_DEV - 1,)),
            pltpu.SemaphoreType.DMA((N_DEV - 1,)),
            pltpu.SemaphoreType.DMA((N_DEV - 1,)),
        ],
        compiler_params=pltpu.CompilerParams(collective_id=0),
    )(x, route_idx, expert_W)


# device time: 11592 ns/iter; 1.8503x vs baseline; 1.2930x over previous
import jax
import jax.numpy as jnp
from jax import lax
from jax.experimental import pallas as pl
from jax.experimental.pallas import tpu as pltpu

N_DEV = 8
EXPERTS_PER_DEV = 2


def kernel(x, router_W, route_idx, expert_W):
    del router_W
    n, d = x.shape
    h = expert_W.shape[-1]
    B = n // N_DEV

    def body(x_ref, idx_ref, ew_ref, out_ref, acc_ref, abf_ref, a2a_ref,
             cstage_ref, b_ssems, b_rsems, c_ssems, c_rsems):
        my = lax.axis_index("i")

        barrier_sem = pltpu.get_barrier_semaphore()
        for j in range(1, N_DEV):
            pl.semaphore_signal(
                barrier_sem, inc=1,
                device_id=((my + j) % N_DEV,),
                device_id_type=pl.DeviceIdType.MESH,
            )

        idx = idx_ref[:, :]
        acc = jnp.zeros((n, h), jnp.float32)
        for e in range(EXPERTS_PER_DEV):
            ge = my * EXPERTS_PER_DEV + e
            y = jnp.dot(x_ref[:, :], ew_ref[e, :, :],
                        preferred_element_type=jnp.float32)
            acc = acc + jnp.where(idx == ge, y, 0.0)
        acc_ref[:, :] = acc
        abf_ref[:, :] = acc.astype(jnp.bfloat16)

        pl.semaphore_wait(barrier_sem, N_DEV - 1)

        bdescs = []
        for j in range(1, N_DEV):
            r = (my + j) % N_DEV
            dsc = pltpu.make_async_remote_copy(
                src_ref=abf_ref.at[pl.ds(r * B, B), :],
                dst_ref=a2a_ref.at[j - 1],
                send_sem=b_ssems.at[j - 1],
                recv_sem=b_rsems.at[j - 1],
                device_id=(r,),
                device_id_type=pl.DeviceIdType.MESH,
            )
            dsc.start()
            bdescs.append(dsc)
        for dsc in bdescs:
            dsc.wait()

        blk = acc_ref[pl.ds(my * B, B), :]
        for j in range(N_DEV - 1):
            blk = blk + a2a_ref[j].astype(jnp.float32)
        cstage_ref[my] = blk.astype(jnp.bfloat16)

        cdescs = []
        for j in range(1, N_DEV):
            r = (my + j) % N_DEV
            dsc = pltpu.make_async_remote_copy(
                src_ref=cstage_ref.at[my],
                dst_ref=cstage_ref.at[my],
                send_sem=c_ssems.at[j - 1],
                recv_sem=c_rsems.at[j - 1],
                device_id=(r,),
                device_id_type=pl.DeviceIdType.MESH,
            )
            dsc.start()
            cdescs.append(dsc)
        for dsc in cdescs:
            dsc.wait()

        out_ref[:, :] = cstage_ref[:, :, :].reshape(n, h).astype(jnp.float32)
        out_ref[pl.ds(my * B, B), :] = blk

    return pl.pallas_call(
        body,
        out_shape=jax.ShapeDtypeStruct((n, h), jnp.float32),
        in_specs=[
            pl.BlockSpec(memory_space=pltpu.VMEM),
            pl.BlockSpec(memory_space=pltpu.VMEM),
            pl.BlockSpec(memory_space=pltpu.VMEM),
        ],
        out_specs=pl.BlockSpec(memory_space=pltpu.VMEM),
        scratch_shapes=[
            pltpu.VMEM((n, h), jnp.float32),
            pltpu.VMEM((n, h), jnp.bfloat16),
            pltpu.VMEM((N_DEV - 1, B, h), jnp.bfloat16),
            pltpu.VMEM((N_DEV, B, h), jnp.bfloat16),
            pltpu.SemaphoreType.DMA((N_DEV - 1,)),
            pltpu.SemaphoreType.DMA((N_DEV - 1,)),
            pltpu.SemaphoreType.DMA((N_DEV - 1,)),
            pltpu.SemaphoreType.DMA((N_DEV - 1,)),
        ],
        compiler_params=pltpu.CompilerParams(collective_id=0),
    )(x, route_idx, expert_W)
